# baseline (speedup 1.0000x reference)
import jax
import jax.numpy as jnp
from jax.experimental import pallas as pl


def _leaky(x):
    return jnp.where(x >= 0, x, 0.01 * x)


def _gru(gi, gh, h):
    d = h.shape[-1]
    i_r, i_z, i_n = gi[:, :d], gi[:, d:2 * d], gi[:, 2 * d:]
    h_r, h_z, h_n = gh[:, :d], gh[:, d:2 * d], gh[:, 2 * d:]
    r = jax.nn.sigmoid(i_r + h_r)
    z = jax.nn.sigmoid(i_z + h_z)
    ng = jnp.tanh(i_n + r * h_n)
    return (1.0 - z) * ng + z * h


def _mp_kernel(obs_ref, adj_ref, w1, b1, w2, b2, w3, b3, mw, mb, wih, whh,
               bih, bhh, feat_ref):
    f32 = jnp.float32
    bf16 = jnp.bfloat16
    obs = obs_ref[0]
    adj = adj_ref[0]

    x = _leaky(jnp.dot(obs, w1[...], preferred_element_type=f32) + b1[...])
    x = _leaky(jnp.dot(x, w2[...], preferred_element_type=f32) + b2[...])
    x = _leaky(jnp.dot(x, w3[...], preferred_element_type=f32) + b3[...])

    mb_v, bih_v, bhh_v = mb[...], bih[...], bhh[...]

    n = adj.shape[0]
    rowsum = jnp.dot(adj, jnp.ones((n, 1), bf16), preferred_element_type=f32)
    msg = rowsum * mb_v
    gi = jnp.dot(jnp.concatenate([x, msg], axis=1), wih[...],
                 preferred_element_type=f32) + bih_v
    gh = jnp.broadcast_to(bhh_v, gi.shape)
    h = _gru(gi, gh, jnp.zeros_like(msg))

    for _ in range(2):
        m = jnp.dot(h, mw[...], preferred_element_type=f32) + mb_v
        msg = jnp.dot(adj, m.astype(bf16), preferred_element_type=f32)
        gi = jnp.dot(jnp.concatenate([x, msg], axis=1), wih[...],
                     preferred_element_type=f32) + bih_v
        gh = jnp.dot(h, whh[...], preferred_element_type=f32) + bhh_v
        h = _gru(gi, gh, h)

    neigh = jnp.dot(adj, h.astype(bf16), preferred_element_type=f32)
    glob = jnp.broadcast_to(jnp.mean(h, axis=0, keepdims=True), h.shape)
    feat_ref[0] = jnp.concatenate([h, neigh, glob], axis=1)


def _readout_kernel(feat_ref, cw, cb, rw, rb, aw, ab,
                    cls_ref, pred_ref, all_ref):
    f32 = jnp.float32
    feat = feat_ref[...]
    cls_ref[...] = jnp.dot(feat, cw[...], preferred_element_type=f32) + cb[...]
    pred_ref[...] = jnp.dot(feat, rw[...], preferred_element_type=f32) + rb[...]
    all_ref[...] = jnp.dot(feat.astype(jnp.bfloat16), aw[...],
                           preferred_element_type=f32) + ab[...]


def kernel(node_obs, node_adj, enc_W1, enc_b1, enc_W2, enc_b2, enc_W3, enc_b3,
           msg_W, msg_b, gru_Wih, gru_Whh, gru_bih, gru_bhh, cls_W, cls_b,
           reg_W, reg_b, all_W, all_b):
    B, N, F = node_obs.shape
    D = enc_W3.shape[0]
    C = cls_W.shape[0]

    mp_args = (
        node_obs, node_adj.astype(jnp.bfloat16),
        enc_W1.T, enc_b1.reshape(1, -1),
        enc_W2.T, enc_b2.reshape(1, -1),
        enc_W3.T, enc_b3.reshape(1, -1),
        msg_W.T, msg_b.reshape(1, -1),
        gru_Wih.T, gru_Whh.T,
        gru_bih.reshape(1, -1), gru_bhh.reshape(1, -1),
    )
    mp_in_specs = [
        pl.BlockSpec((1, N, F), lambda b: (b, 0, 0)),
        pl.BlockSpec((1, N, N), lambda b: (b, 0, 0)),
    ] + [
        pl.BlockSpec(a.shape, lambda b, nd=a.ndim: (0,) * nd)
        for a in mp_args[2:]
    ]
    feat = pl.pallas_call(
        _mp_kernel,
        grid=(B,),
        in_specs=mp_in_specs,
        out_specs=pl.BlockSpec((1, N, 3 * D), lambda b: (b, 0, 0)),
        out_shape=jax.ShapeDtypeStruct((B, N, 3 * D), node_obs.dtype),
    )(*mp_args)

    R = 512
    flat = feat.reshape(B * N, 3 * D)
    ro_args = (
        flat,
        cls_W.T, cls_b.reshape(1, -1),
        reg_W.T, reg_b.reshape(1, -1),
        all_W.T.astype(jnp.bfloat16), all_b.reshape(1, -1),
    )
    ro_in_specs = [
        pl.BlockSpec((R, 3 * D), lambda i: (i, 0)),
    ] + [
        pl.BlockSpec(a.shape, lambda i, nd=a.ndim: (0,) * nd)
        for a in ro_args[1:]
    ]
    cls, pred, pred_all = pl.pallas_call(
        _readout_kernel,
        grid=(B * N // R,),
        in_specs=ro_in_specs,
        out_specs=(
            pl.BlockSpec((R, C), lambda i: (i, 0)),
            pl.BlockSpec((R, 1), lambda i: (i, 0)),
            pl.BlockSpec((R, N), lambda i: (i, 0)),
        ),
        out_shape=(
            jax.ShapeDtypeStruct((B * N, C), node_obs.dtype),
            jax.ShapeDtypeStruct((B * N, 1), node_obs.dtype),
            jax.ShapeDtypeStruct((B * N, N), node_obs.dtype),
        ),
    )(*ro_args)

    return (cls.reshape(B, N, C), pred.reshape(B, N, 1),
            pred_all.reshape(B, N, N))


# speedup vs baseline: 1.2008x; 1.2008x over previous
import jax
import jax.numpy as jnp
from jax.experimental import pallas as pl


def _leaky(x):
    return jnp.where(x >= 0, x, 0.01 * x)


def _gru(gi, gh, h):
    d = h.shape[-1]
    i_r, i_z, i_n = gi[:, :d], gi[:, d:2 * d], gi[:, 2 * d:]
    h_r, h_z, h_n = gh[:, :d], gh[:, d:2 * d], gh[:, 2 * d:]
    r = jax.nn.sigmoid(i_r + h_r)
    z = jax.nn.sigmoid(i_z + h_z)
    ng = jnp.tanh(i_n + r * h_n)
    return (1.0 - z) * ng + z * h


def _mp_kernel(obs_ref, adj_ref, w1, b1, w2, b2, w3, b3, mw, mb, wih, whh,
               bih, bhh, feat_ref):
    f32 = jnp.float32
    bf16 = jnp.bfloat16
    obs = obs_ref[0]
    adj = adj_ref[0].astype(bf16)

    x = _leaky(jnp.dot(obs, w1[...], preferred_element_type=f32) + b1[...])
    x = _leaky(jnp.dot(x, w2[...], preferred_element_type=f32) + b2[...])
    x = _leaky(jnp.dot(x, w3[...], preferred_element_type=f32) + b3[...])

    mb_v, bih_v, bhh_v = mb[...], bih[...], bhh[...]

    n = adj.shape[0]
    rowsum = jnp.dot(adj, jnp.ones((n, 1), bf16), preferred_element_type=f32)
    msg = rowsum * mb_v
    gi = jnp.dot(jnp.concatenate([x, msg], axis=1), wih[...],
                 preferred_element_type=f32) + bih_v
    gh = jnp.broadcast_to(bhh_v, gi.shape)
    h = _gru(gi, gh, jnp.zeros_like(msg))

    for _ in range(2):
        m = jnp.dot(h, mw[...], preferred_element_type=f32) + mb_v
        msg = jnp.dot(adj, m.astype(bf16), preferred_element_type=f32)
        gi = jnp.dot(jnp.concatenate([x, msg], axis=1), wih[...],
                     preferred_element_type=f32) + bih_v
        gh = jnp.dot(h, whh[...], preferred_element_type=f32) + bhh_v
        h = _gru(gi, gh, h)

    neigh = jnp.dot(adj, h.astype(bf16), preferred_element_type=f32)
    glob = jnp.broadcast_to(jnp.mean(h, axis=0, keepdims=True), h.shape)
    feat_ref[0] = jnp.concatenate([h, neigh, glob], axis=1)


def _readout_kernel(feat_ref, cw, cb, rw, rb, aw, ab,
                    cls_ref, pred_ref, all_ref):
    f32 = jnp.float32
    feat = feat_ref[...]
    cls_ref[...] = jnp.dot(feat, cw[...], preferred_element_type=f32) + cb[...]
    pred_ref[...] = jnp.dot(feat, rw[...], preferred_element_type=f32) + rb[...]
    all_ref[...] = jnp.dot(feat.astype(jnp.bfloat16), aw[...],
                           preferred_element_type=f32) + ab[...]


def kernel(node_obs, node_adj, enc_W1, enc_b1, enc_W2, enc_b2, enc_W3, enc_b3,
           msg_W, msg_b, gru_Wih, gru_Whh, gru_bih, gru_bhh, cls_W, cls_b,
           reg_W, reg_b, all_W, all_b):
    B, N, F = node_obs.shape
    D = enc_W3.shape[0]
    C = cls_W.shape[0]

    mp_args = (
        node_obs, node_adj,
        enc_W1.T, enc_b1.reshape(1, -1),
        enc_W2.T, enc_b2.reshape(1, -1),
        enc_W3.T, enc_b3.reshape(1, -1),
        msg_W.T, msg_b.reshape(1, -1),
        gru_Wih.T, gru_Whh.T,
        gru_bih.reshape(1, -1), gru_bhh.reshape(1, -1),
    )
    mp_in_specs = [
        pl.BlockSpec((1, N, F), lambda b: (b, 0, 0)),
        pl.BlockSpec((1, N, N), lambda b: (b, 0, 0)),
    ] + [
        pl.BlockSpec(a.shape, lambda b, nd=a.ndim: (0,) * nd)
        for a in mp_args[2:]
    ]
    feat = pl.pallas_call(
        _mp_kernel,
        grid=(B,),
        in_specs=mp_in_specs,
        out_specs=pl.BlockSpec((1, N, 3 * D), lambda b: (b, 0, 0)),
        out_shape=jax.ShapeDtypeStruct((B, N, 3 * D), node_obs.dtype),
    )(*mp_args)

    R = 512
    flat = feat.reshape(B * N, 3 * D)
    ro_args = (
        flat,
        cls_W.T, cls_b.reshape(1, -1),
        reg_W.T, reg_b.reshape(1, -1),
        all_W.T.astype(jnp.bfloat16), all_b.reshape(1, -1),
    )
    ro_in_specs = [
        pl.BlockSpec((R, 3 * D), lambda i: (i, 0)),
    ] + [
        pl.BlockSpec(a.shape, lambda i, nd=a.ndim: (0,) * nd)
        for a in ro_args[1:]
    ]
    cls, pred, pred_all = pl.pallas_call(
        _readout_kernel,
        grid=(B * N // R,),
        in_specs=ro_in_specs,
        out_specs=(
            pl.BlockSpec((R, C), lambda i: (i, 0)),
            pl.BlockSpec((R, 1), lambda i: (i, 0)),
            pl.BlockSpec((R, N), lambda i: (i, 0)),
        ),
        out_shape=(
            jax.ShapeDtypeStruct((B * N, C), node_obs.dtype),
            jax.ShapeDtypeStruct((B * N, 1), node_obs.dtype),
            jax.ShapeDtypeStruct((B * N, N), node_obs.dtype),
        ),
    )(*ro_args)

    return (cls.reshape(B, N, C), pred.reshape(B, N, 1),
            pred_all.reshape(B, N, N))


# speedup vs baseline: 1.3673x; 1.1386x over previous
import jax
import jax.numpy as jnp
from jax.experimental import pallas as pl


def _leaky(x):
    return jnp.where(x >= 0, x, 0.01 * x)


def _gru(gi, gh, h):
    d = h.shape[-1]
    i_r, i_z, i_n = gi[:, :d], gi[:, d:2 * d], gi[:, 2 * d:]
    h_r, h_z, h_n = gh[:, :d], gh[:, d:2 * d], gh[:, 2 * d:]
    r = jax.nn.sigmoid(i_r + h_r)
    z = jax.nn.sigmoid(i_z + h_z)
    ng = jnp.tanh(i_n + r * h_n)
    return (1.0 - z) * ng + z * h


def _mp_kernel(obs_ref, adj_ref, w1, b1, w2, b2, w3, b3, mw, mb, wih_x, wih_m,
               whh, bih, bhh, feat_ref):
    f32 = jnp.float32
    bf16 = jnp.bfloat16
    obs = obs_ref[0]
    adj = adj_ref[0]

    x = _leaky(jnp.dot(obs.astype(bf16), w1[...],
                       preferred_element_type=f32) + b1[...])
    x = _leaky(jnp.dot(x.astype(bf16), w2[...],
                       preferred_element_type=f32) + b2[...])
    x = _leaky(jnp.dot(x.astype(bf16), w3[...],
                       preferred_element_type=f32) + b3[...])

    mb_v, bih_v, bhh_v = mb[...], bih[...], bhh[...]

    rowsum = jnp.sum(adj, axis=1, keepdims=True)
    adj_bf = adj.astype(bf16)
    x_bf = x.astype(bf16)
    gi_x = jnp.dot(x_bf, wih_x[...], preferred_element_type=f32) + bih_v
    msg = rowsum * mb_v
    gi = gi_x + jnp.dot(msg, wih_m[...], preferred_element_type=f32)
    gh = jnp.broadcast_to(bhh_v, gi.shape)
    h = _gru(gi, gh, jnp.zeros_like(msg))

    for _ in range(2):
        m = jnp.dot(h.astype(bf16), mw[...],
                    preferred_element_type=f32) + mb_v
        msg = jnp.dot(adj_bf, m.astype(bf16), preferred_element_type=f32)
        gi = gi_x + jnp.dot(msg, wih_m[...], preferred_element_type=f32)
        gh = jnp.dot(h.astype(bf16), whh[...],
                     preferred_element_type=f32) + bhh_v
        h = _gru(gi, gh, h)

    neigh = jnp.dot(adj_bf, h.astype(bf16), preferred_element_type=f32)
    glob = jnp.broadcast_to(jnp.mean(h, axis=0, keepdims=True), h.shape)
    feat_ref[0] = jnp.concatenate([h, neigh, glob], axis=1)


def _readout_kernel(feat_ref, cw, cb, rw, rb, aw, ab,
                    cls_ref, pred_ref, all_ref):
    f32 = jnp.float32
    feat = feat_ref[...]
    cls_ref[...] = jnp.dot(feat, cw[...], preferred_element_type=f32) + cb[...]
    pred_ref[...] = jnp.dot(feat, rw[...], preferred_element_type=f32) + rb[...]
    all_ref[...] = jnp.dot(feat, aw[...], preferred_element_type=f32) + ab[...]


def kernel(node_obs, node_adj, enc_W1, enc_b1, enc_W2, enc_b2, enc_W3, enc_b3,
           msg_W, msg_b, gru_Wih, gru_Whh, gru_bih, gru_bhh, cls_W, cls_b,
           reg_W, reg_b, all_W, all_b):
    B, N, F = node_obs.shape
    D = enc_W3.shape[0]
    C = cls_W.shape[0]

    bf16 = jnp.bfloat16
    mp_args = (
        node_obs, node_adj,
        enc_W1.T.astype(bf16), enc_b1.reshape(1, -1),
        enc_W2.T.astype(bf16), enc_b2.reshape(1, -1),
        enc_W3.T.astype(bf16), enc_b3.reshape(1, -1),
        msg_W.T.astype(bf16), msg_b.reshape(1, -1),
        gru_Wih.T[:D].astype(bf16), gru_Wih.T[D:],
        gru_Whh.T.astype(bf16),
        gru_bih.reshape(1, -1), gru_bhh.reshape(1, -1),
    )
    mp_in_specs = [
        pl.BlockSpec((1, N, F), lambda b: (b, 0, 0)),
        pl.BlockSpec((1, N, N), lambda b: (b, 0, 0)),
    ] + [
        pl.BlockSpec(a.shape, lambda b, nd=a.ndim: (0,) * nd)
        for a in mp_args[2:]
    ]
    feat = pl.pallas_call(
        _mp_kernel,
        grid=(B,),
        in_specs=mp_in_specs,
        out_specs=pl.BlockSpec((1, N, 3 * D), lambda b: (b, 0, 0)),
        out_shape=jax.ShapeDtypeStruct((B, N, 3 * D), node_obs.dtype),
    )(*mp_args)

    R = 512
    flat = feat.reshape(B * N, 3 * D)
    ro_args = (
        flat,
        cls_W.T, cls_b.reshape(1, -1),
        reg_W.T, reg_b.reshape(1, -1),
        all_W.T, all_b.reshape(1, -1),
    )
    ro_in_specs = [
        pl.BlockSpec((R, 3 * D), lambda i: (i, 0)),
    ] + [
        pl.BlockSpec(a.shape, lambda i, nd=a.ndim: (0,) * nd)
        for a in ro_args[1:]
    ]
    cls, pred, pred_all = pl.pallas_call(
        _readout_kernel,
        grid=(B * N // R,),
        in_specs=ro_in_specs,
        out_specs=(
            pl.BlockSpec((R, C), lambda i: (i, 0)),
            pl.BlockSpec((R, 1), lambda i: (i, 0)),
            pl.BlockSpec((R, N), lambda i: (i, 0)),
        ),
        out_shape=(
            jax.ShapeDtypeStruct((B * N, C), node_obs.dtype),
            jax.ShapeDtypeStruct((B * N, 1), node_obs.dtype),
            jax.ShapeDtypeStruct((B * N, N), node_obs.dtype),
        ),
    )(*ro_args)

    return (cls.reshape(B, N, C), pred.reshape(B, N, 1),
            pred_all.reshape(B, N, N))


# speedup vs baseline: 1.5713x; 1.1491x over previous
import jax
import jax.numpy as jnp
from jax import lax
from jax.experimental import pallas as pl

_NT = (((1,), (1,)), ((), ()))
_TN = (((0,), (0,)), ((), ()))


def _gru_t(gi, gh, h):
    d = h.shape[0]
    i_r, i_z, i_n = gi[:d], gi[d:2 * d], gi[2 * d:]
    h_r, h_z, h_n = gh[:d], gh[d:2 * d], gh[2 * d:]
    r = jax.nn.sigmoid(i_r + h_r)
    z = jax.nn.sigmoid(i_z + h_z)
    ng = jnp.tanh(i_n + r * h_n)
    return (1.0 - z) * ng + z * h


def _mp_kernel(obs_ref, adj_ref, w1, b1, w2, b2, w3, b3, mw, mb, wih_x, wih_m,
               whh, bih, bhh, feat_ref):
    f32 = jnp.float32

    def leaky(v):
        return jnp.where(v >= 0, v, 0.01 * v)

    obs = obs_ref[...]
    adj = adj_ref[...]
    n = adj.shape[0]

    xt = leaky(lax.dot_general(w1[...], obs, _NT,
                               preferred_element_type=f32) + b1[...])
    xt = leaky(jnp.dot(w2[...], xt, preferred_element_type=f32) + b2[...])
    xt = leaky(jnp.dot(w3[...], xt, preferred_element_type=f32) + b3[...])

    mb_v, bih_v, bhh_v = mb[...], bih[...], bhh[...]

    gi_x = jnp.dot(wih_x[...], xt, preferred_element_type=f32) + bih_v

    rowsum_t = lax.dot_general(jnp.ones((1, n), f32), adj, _NT,
                               preferred_element_type=f32)
    msg_t = mb_v * rowsum_t
    gi = gi_x + jnp.dot(wih_m[...], msg_t, preferred_element_type=f32)
    gh = jnp.broadcast_to(bhh_v, gi.shape)
    h = _gru_t(gi, gh, jnp.zeros_like(msg_t))

    for _ in range(2):
        m_t = jnp.dot(mw[...], h, preferred_element_type=f32) + mb_v
        msg_t = lax.dot_general(m_t, adj, _NT, preferred_element_type=f32)
        gi = gi_x + jnp.dot(wih_m[...], msg_t, preferred_element_type=f32)
        gh = jnp.dot(whh[...], h, preferred_element_type=f32) + bhh_v
        h = _gru_t(gi, gh, h)

    neigh_t = lax.dot_general(h, adj, _NT, preferred_element_type=f32)
    glob_t = jnp.broadcast_to(jnp.mean(h, axis=1, keepdims=True), h.shape)
    feat_ref[...] = jnp.concatenate([h, neigh_t, glob_t], axis=0)


def _readout_kernel(feat_ref, cw, cb, rw, rb, aw, ab,
                    cls_ref, pred_ref, all_ref):
    f32 = jnp.float32
    ft = feat_ref[...]
    cls_ref[...] = lax.dot_general(ft, cw[...], _TN,
                                   preferred_element_type=f32) + cb[...]
    pred_ref[...] = lax.dot_general(ft, rw[...], _TN,
                                    preferred_element_type=f32) + rb[...]
    all_ref[...] = lax.dot_general(ft, aw[...], _TN,
                                   preferred_element_type=f32) + ab[...]


def kernel(node_obs, node_adj, enc_W1, enc_b1, enc_W2, enc_b2, enc_W3, enc_b3,
           msg_W, msg_b, gru_Wih, gru_Whh, gru_bih, gru_bhh, cls_W, cls_b,
           reg_W, reg_b, all_W, all_b):
    B, N, F = node_obs.shape
    D = enc_W3.shape[0]
    C = cls_W.shape[0]

    mp_args = (
        node_obs, node_adj,
        enc_W1, enc_b1.reshape(-1, 1),
        enc_W2, enc_b2.reshape(-1, 1),
        enc_W3, enc_b3.reshape(-1, 1),
        msg_W, msg_b.reshape(-1, 1),
        gru_Wih[:, :D], gru_Wih[:, D:],
        gru_Whh,
        gru_bih.reshape(-1, 1), gru_bhh.reshape(-1, 1),
    )
    mp_in_specs = [
        pl.BlockSpec((None, N, F), lambda b: (b, 0, 0)),
        pl.BlockSpec((None, N, N), lambda b: (b, 0, 0)),
    ] + [
        pl.BlockSpec(a.shape, lambda b, nd=a.ndim: (0,) * nd)
        for a in mp_args[2:]
    ]
    feat_t = pl.pallas_call(
        _mp_kernel,
        grid=(B,),
        in_specs=mp_in_specs,
        out_specs=pl.BlockSpec((None, 3 * D, N), lambda b: (b, 0, 0)),
        out_shape=jax.ShapeDtypeStruct((B, 3 * D, N), node_obs.dtype),
    )(*mp_args)

    R = 512
    ro_args = (
        feat_t,
        cls_W.T, cls_b.reshape(1, -1),
        reg_W.T, reg_b.reshape(1, -1),
        all_W.T, all_b.reshape(1, -1),
    )
    ro_in_specs = [
        pl.BlockSpec((None, 3 * D, R), lambda b, j: (b, 0, j)),
    ] + [
        pl.BlockSpec(a.shape, lambda b, j, nd=a.ndim: (0,) * nd)
        for a in ro_args[1:]
    ]
    cls, pred, pred_all = pl.pallas_call(
        _readout_kernel,
        grid=(B, N // R),
        in_specs=ro_in_specs,
        out_specs=(
            pl.BlockSpec((None, R, C), lambda b, j: (b, j, 0)),
            pl.BlockSpec((None, R, 1), lambda b, j: (b, j, 0)),
            pl.BlockSpec((None, R, N), lambda b, j: (b, j, 0)),
        ),
        out_shape=(
            jax.ShapeDtypeStruct((B, N, C), node_obs.dtype),
            jax.ShapeDtypeStruct((B, N, 1), node_obs.dtype),
            jax.ShapeDtypeStruct((B, N, N), node_obs.dtype),
        ),
    )(*ro_args)

    return (cls, pred, pred_all)
